# baseline (speedup 1.0000x reference)
import jax
import jax.numpy as jnp
from jax import lax
from jax.experimental import pallas as pl


def _one_batch(x, w, w_norm, lane_k, sub_k):
    xt = x.T
    K = w.shape[0]
    x_norm = jnp.sum(xt * xt, axis=1, keepdims=True)
    dots = lax.dot_general(xt, w, (((1,), (1,)), ((), ())),
                           preferred_element_type=jnp.float32)
    dist = (x_norm + w_norm[None, :]) - 2.0 * dots
    m = jnp.min(dist, axis=1, keepdims=True)
    kidx = jnp.min(jnp.where(dist == m, lane_k, float(K)), axis=1,
                   keepdims=True)
    return (sub_k == kidx.T).astype(jnp.float32)


def _som_body(inp_ref, w_ref, out_ref):
    w = w_ref[...]
    K = w.shape[0]
    T = inp_ref.shape[2]
    w_norm = jnp.sum(w * w, axis=1)
    lane_k = lax.broadcasted_iota(jnp.int32, (1, K), 1).astype(jnp.float32)
    sub_k = lax.broadcasted_iota(jnp.int32, (K, 1), 0).astype(jnp.float32)
    xt = jnp.concatenate([inp_ref[j].T for j in range(8)], axis=0)
    x_norm = jnp.sum(xt * xt, axis=1, keepdims=True)
    dots = lax.dot_general(xt, w, (((1,), (1,)), ((), ())),
                           preferred_element_type=jnp.float32)
    dist = (x_norm + w_norm[None, :]) - 2.0 * dots
    m = jnp.min(dist, axis=1, keepdims=True)
    kidx = jnp.min(jnp.where(dist == m, lane_k, float(K)), axis=1,
                   keepdims=True)
    oh = (sub_k == kidx.T).astype(jnp.float32)
    for j in range(8):
        out_ref[j, 0] = oh[:, j * T:(j + 1) * T]


def kernel(inp, W):
    B, C, T = inp.shape
    K = W.shape[0]
    return pl.pallas_call(
        _som_body,
        grid=(B // 8,),
        in_specs=[
            pl.BlockSpec((8, C, T), lambda i: (i, 0, 0)),
            pl.BlockSpec((K, C), lambda i: (0, 0)),
        ],
        out_specs=pl.BlockSpec((8, 1, K, T), lambda i: (i, 0, 0, 0)),
        out_shape=jax.ShapeDtypeStruct((B, 1, K, T), jnp.float32),
    )(inp, W)


# speedup vs baseline: 1.0433x; 1.0433x over previous
import jax
import jax.numpy as jnp
from jax import lax
from jax.experimental import pallas as pl


def _som_body(inp_ref, w_ref, out_ref):
    w = w_ref[...]
    K = w.shape[0]
    T = inp_ref.shape[2]
    w_norm = jnp.sum(w * w, axis=1)
    lane_k = lax.broadcasted_iota(jnp.int32, (1, K), 1).astype(jnp.float32)
    sub_k = lax.broadcasted_iota(jnp.int32, (K, 1), 0).astype(jnp.float32)
    xt = jnp.concatenate([inp_ref[0].T, inp_ref[1].T,
                          inp_ref[2].T, inp_ref[3].T], axis=0)
    x_norm = jnp.sum(xt * xt, axis=1, keepdims=True)
    dots = lax.dot_general(xt, w, (((1,), (1,)), ((), ())),
                           preferred_element_type=jnp.float32)
    dist = (x_norm + w_norm[None, :]) - 2.0 * dots
    m = jnp.min(dist, axis=1, keepdims=True)
    kidx = jnp.min(jnp.where(dist == m, lane_k, float(K)), axis=1,
                   keepdims=True)
    oh = (sub_k == kidx.T).astype(jnp.float32)
    out_ref[0, 0] = oh[:, :T]
    out_ref[1, 0] = oh[:, T:2 * T]
    out_ref[2, 0] = oh[:, 2 * T:3 * T]
    out_ref[3, 0] = oh[:, 3 * T:]


def kernel(inp, W):
    B, C, T = inp.shape
    K = W.shape[0]
    return pl.pallas_call(
        _som_body,
        grid=(B // 4,),
        in_specs=[
            pl.BlockSpec((4, C, T), lambda i: (i, 0, 0)),
            pl.BlockSpec((K, C), lambda i: (0, 0)),
        ],
        out_specs=pl.BlockSpec((4, 1, K, T), lambda i: (i, 0, 0, 0)),
        out_shape=jax.ShapeDtypeStruct((B, 1, K, T), jnp.float32),
    )(inp, W)


# speedup vs baseline: 1.0672x; 1.0229x over previous
import jax
import jax.numpy as jnp
from jax import lax
from jax.experimental import pallas as pl


def _som_body(inp_ref, w_ref, out_ref):
    w = w_ref[...]
    K = w.shape[0]
    T = inp_ref.shape[2]
    w_norm = jnp.sum(w * w, axis=1)
    lane_k = lax.broadcasted_iota(jnp.int32, (1, K), 1).astype(jnp.float32)
    sub_k = lax.broadcasted_iota(jnp.int32, (K, 1), 0).astype(jnp.float32)
    def pair(a, b):
        xt = jnp.concatenate([inp_ref[a].T, inp_ref[b].T], axis=0)
        x_norm = jnp.sum(xt * xt, axis=1, keepdims=True)
        dots = lax.dot_general(xt, w, (((1,), (1,)), ((), ())),
                               preferred_element_type=jnp.float32)
        dist = (x_norm + w_norm[None, :]) - 2.0 * dots
        m = jnp.min(dist, axis=1, keepdims=True)
        kidx = jnp.min(jnp.where(dist == m, lane_k, float(K)), axis=1,
                       keepdims=True)
        return (sub_k == kidx.T).astype(jnp.float32)

    oh0 = pair(0, 1)
    oh1 = pair(2, 3)
    out_ref[0, 0] = oh0[:, :T]
    out_ref[1, 0] = oh0[:, T:]
    out_ref[2, 0] = oh1[:, :T]
    out_ref[3, 0] = oh1[:, T:]


def kernel(inp, W):
    B, C, T = inp.shape
    K = W.shape[0]
    return pl.pallas_call(
        _som_body,
        grid=(B // 4,),
        in_specs=[
            pl.BlockSpec((4, C, T), lambda i: (i, 0, 0)),
            pl.BlockSpec((K, C), lambda i: (0, 0)),
        ],
        out_specs=pl.BlockSpec((4, 1, K, T), lambda i: (i, 0, 0, 0)),
        out_shape=jax.ShapeDtypeStruct((B, 1, K, T), jnp.float32),
    )(inp, W)
